# Initial kernel scaffold; baseline (speedup 1.0000x reference)
import jax
import jax.numpy as jnp
from jax.experimental import pallas as pl


def kernel(x, edge_index, W1, b1, W2, b2):
    raise NotImplementedError("write your pallas kernel here")


# speedup vs baseline: 8.5455x; 8.5455x over previous
import jax
import jax.numpy as jnp
from jax import lax
from jax.experimental import pallas as pl
from jax.experimental.pallas import tpu as pltpu
from jax.experimental.pallas import tpu_sc as plsc

NC = 2
NS = 16
NW = NC * NS
LANE = 16
CHUNK = 128
DEGW = 16
NB = 4
KD = 8


def _mesh():
    return plsc.VectorSubcoreMesh(
        core_axis_name="c", subcore_axis_name="s", num_cores=NC, num_subcores=NS
    )


def _deg_body(dst3, out, idx_v, ones_v, zrow_v, acc, dsem):
    c = lax.axis_index("c")
    s = lax.axis_index("s")
    w = c * NS + s
    chw = idx_v.shape[0]
    rpt = zrow_v.shape[0]

    pltpu.sync_copy(dst3.at[w], idx_v)

    @pl.loop(0, CHUNK)
    def _(r):
        ones_v[r, :] = jnp.ones((DEGW,), jnp.float32)

    @pl.loop(0, rpt)
    def _(r):
        zrow_v[r, :] = jnp.zeros((DEGW,), jnp.float32)

    pltpu.sync_copy(zrow_v, acc.at[pl.ds(s * rpt, rpt)])
    plsc.subcore_barrier()

    for j in range(KD):
        pltpu.async_copy(ones_v, acc.at[idx_v.at[j]], dsem, add=True)

    @pl.loop(0, chw - KD)
    def _(j):
        pltpu.async_copy(ones_v, acc.at[idx_v.at[j + KD]], dsem, add=True)
        pltpu.make_async_copy(ones_v, acc.at[idx_v.at[0]], dsem).wait()

    for _j in range(KD):
        pltpu.make_async_copy(ones_v, acc.at[idx_v.at[0]], dsem).wait()

    plsc.subcore_barrier()
    pltpu.sync_copy(acc.at[pl.ds(s * rpt, rpt)], out.at[c, pl.ds(s * rpt, rpt)])


def _agg_body(y_lo, y_hi, src3, dst3, out_lo, out_hi, idxs_v, idxd_v, rows_v,
              acc, g0, g1, g2, g3, s0, s1, s2, s3):
    gsems = (g0, g1, g2, g3)
    ssems = (s0, s1, s2, s3)
    c = lax.axis_index("c")
    s = lax.axis_index("s")
    w = c * NS + s
    chw = idxs_v.shape[0]
    n_acc, dh = acc.shape
    rpt = n_acc // NS

    pltpu.sync_copy(src3.at[w], idxs_v)
    pltpu.sync_copy(dst3.at[w], idxd_v)

    def zero_acc_slice():
        for k in range(rpt // CHUNK):
            pltpu.sync_copy(rows_v.at[0], acc.at[pl.ds(s * rpt + k * CHUNK, CHUNK)])

    @pl.loop(0, CHUNK)
    def _(r):
        for q in range(dh // LANE):
            rows_v[0, r, pl.ds(q * LANE, LANE)] = jnp.zeros((LANE,), jnp.float32)

    def run_half(y, out):
        def start_gather(j, b):
            pltpu.async_copy(y.at[idxs_v.at[j]], rows_v.at[b], gsems[b])

        def wait_gather(b):
            pltpu.make_async_copy(y.at[idxs_v.at[0]], rows_v.at[b], gsems[b]).wait()

        def start_scatter(j, b):
            pltpu.async_copy(rows_v.at[b], acc.at[idxd_v.at[j]], ssems[b], add=True)

        def wait_scatter(b):
            pltpu.make_async_copy(
                rows_v.at[b], acc.at[idxd_v.at[0]], ssems[b]).wait()

        for b in range(NB):
            start_gather(b, b)

        @pl.loop(0, chw // NB - 1)
        def _(g):
            base = g * NB
            for b in range(NB):
                wait_gather(b)
                start_scatter(base + b, b)
            for b in range(NB):
                wait_scatter(b)
                start_gather(base + NB + b, b)

        last = chw - NB
        for b in range(NB):
            wait_gather(b)
            start_scatter(last + b, b)
        for b in range(NB):
            wait_scatter(b)

        plsc.subcore_barrier()
        for k in range(rpt // CHUNK):
            off = s * rpt + k * CHUNK
            pltpu.sync_copy(acc.at[pl.ds(off, CHUNK)], out.at[c, pl.ds(off, CHUNK)])

    zero_acc_slice()
    plsc.subcore_barrier()
    run_half(y_lo, out_lo)

    @pl.loop(0, CHUNK)
    def _(r):
        for q in range(dh // LANE):
            rows_v[0, r, pl.ds(q * LANE, LANE)] = jnp.zeros((LANE,), jnp.float32)

    zero_acc_slice()
    plsc.subcore_barrier()
    run_half(y_hi, out_hi)


_SC_PARAMS = pltpu.CompilerParams(use_tc_tiling_on_sc=False)


def _count_degrees(dst3, n_acc):
    chw = dst3.shape[1]
    rpt = n_acc // NS
    return pl.kernel(
        _deg_body,
        out_type=jax.ShapeDtypeStruct((NC, n_acc, DEGW), jnp.float32),
        mesh=_mesh(),
        compiler_params=_SC_PARAMS,
        scratch_types=[
            pltpu.VMEM((chw, CHUNK), jnp.int32),
            pltpu.VMEM((CHUNK, DEGW), jnp.float32),
            pltpu.VMEM((rpt, DEGW), jnp.float32),
            pltpu.VMEM_SHARED((n_acc, DEGW), jnp.float32),
            pltpu.SemaphoreType.DMA,
        ],
    )(dst3)


def _aggregate(y_lo, y_hi, src3, dst3, n_acc):
    chw = src3.shape[1]
    dh = y_lo.shape[1]
    out_t = jax.ShapeDtypeStruct((NC, n_acc, dh), jnp.float32)
    return pl.kernel(
        _agg_body,
        out_type=[out_t, out_t],
        mesh=_mesh(),
        compiler_params=_SC_PARAMS,
        scratch_types=[
            pltpu.VMEM((chw, CHUNK), jnp.int32),
            pltpu.VMEM((chw, CHUNK), jnp.int32),
            pltpu.VMEM((NB, CHUNK, dh), jnp.float32),
            pltpu.VMEM_SHARED((n_acc, dh), jnp.float32),
        ] + [pltpu.SemaphoreType.DMA] * (2 * NB),
    )(y_lo, y_hi, src3, dst3)


def _row_block(n):
    for bm in (1024, 1000, 800, 640, 512, 400, 256, 200, 128, 80, 40, 8):
        if n % bm == 0:
            return bm
    return n


def _first_layer(x, w1, degs):
    n, d_in = x.shape
    d_h = w1.shape[1]
    dh2 = d_h // 2
    bm = _row_block(n)

    def body(deg_ref, x_ref, w_ref, ylo_ref, yhi_ref, dinv_ref):
        deg = deg_ref[0, :, 0:1] + deg_ref[1, :, 0:1] + 1.0
        dinvb = jnp.broadcast_to(lax.rsqrt(deg), (bm, d_h))
        dinv_ref[...] = dinvb
        y = jnp.dot(
            x_ref[...], w_ref[...],
            preferred_element_type=jnp.float32,
            precision=lax.Precision.HIGHEST,
        ) * dinvb
        ylo_ref[...] = y[:, :dh2]
        yhi_ref[...] = y[:, dh2:]

    return pl.pallas_call(
        body,
        grid=(n // bm,),
        in_specs=[
            pl.BlockSpec((NC, bm, DEGW), lambda i: (0, i, 0)),
            pl.BlockSpec((bm, d_in), lambda i: (i, 0)),
            pl.BlockSpec((d_in, d_h), lambda i: (0, 0)),
        ],
        out_specs=[
            pl.BlockSpec((bm, dh2), lambda i: (i, 0)),
            pl.BlockSpec((bm, dh2), lambda i: (i, 0)),
            pl.BlockSpec((bm, d_h), lambda i: (i, 0)),
        ],
        out_shape=[
            jax.ShapeDtypeStruct((n, dh2), jnp.float32),
            jax.ShapeDtypeStruct((n, dh2), jnp.float32),
            jax.ShapeDtypeStruct((n, d_h), jnp.float32),
        ],
    )(degs, x, w1)


def _mid_layer(alo, ahi, ylo, yhi, dinvb, b1, w2):
    n, d = dinvb.shape
    dh2 = d // 2
    bm = _row_block(n)

    def body(alo_ref, ahi_ref, ylo_ref, yhi_ref, dinv_ref, b_ref, w_ref,
             y2lo_ref, y2hi_ref):
        dinv = dinv_ref[...]
        h_lo = dinv[:, :dh2] * (alo_ref[0] + alo_ref[1] + ylo_ref[...]) \
            + b_ref[:, :dh2]
        h_hi = dinv[:, dh2:] * (ahi_ref[0] + ahi_ref[1] + yhi_ref[...]) \
            + b_ref[:, dh2:]
        h = jnp.maximum(jnp.concatenate([h_lo, h_hi], axis=-1), 0.0)
        y2 = jnp.dot(
            h, w_ref[...],
            preferred_element_type=jnp.float32,
            precision=lax.Precision.HIGHEST,
        ) * dinv
        y2lo_ref[...] = y2[:, :dh2]
        y2hi_ref[...] = y2[:, dh2:]

    half = pl.BlockSpec((bm, dh2), lambda i: (i, 0))
    part = pl.BlockSpec((NC, bm, dh2), lambda i: (0, i, 0))
    return pl.pallas_call(
        body,
        grid=(n // bm,),
        in_specs=[
            part, part, half, half,
            pl.BlockSpec((bm, d), lambda i: (i, 0)),
            pl.BlockSpec((1, d), lambda i: (0, 0)),
            pl.BlockSpec((d, d), lambda i: (0, 0)),
        ],
        out_specs=[half, half],
        out_shape=[
            jax.ShapeDtypeStruct((n, dh2), jnp.float32),
            jax.ShapeDtypeStruct((n, dh2), jnp.float32),
        ],
    )(alo, ahi, ylo, yhi, dinvb, b1, w2)


def _final_layer(alo, ahi, ylo, yhi, dinvb, b2):
    n, d = dinvb.shape
    dh2 = d // 2
    bm = _row_block(n)

    def body(alo_ref, ahi_ref, ylo_ref, yhi_ref, dinv_ref, b_ref, h_ref):
        dinv = dinv_ref[...]
        h_lo = dinv[:, :dh2] * (alo_ref[0] + alo_ref[1] + ylo_ref[...]) \
            + b_ref[:, :dh2]
        h_hi = dinv[:, dh2:] * (ahi_ref[0] + ahi_ref[1] + yhi_ref[...]) \
            + b_ref[:, dh2:]
        h_ref[...] = jnp.maximum(jnp.concatenate([h_lo, h_hi], axis=-1), 0.0)

    half = pl.BlockSpec((bm, dh2), lambda i: (i, 0))
    part = pl.BlockSpec((NC, bm, dh2), lambda i: (0, i, 0))
    return pl.pallas_call(
        body,
        grid=(n // bm,),
        in_specs=[
            part, part, half, half,
            pl.BlockSpec((bm, d), lambda i: (i, 0)),
            pl.BlockSpec((1, d), lambda i: (0, 0)),
        ],
        out_specs=pl.BlockSpec((bm, d), lambda i: (i, 0)),
        out_shape=jax.ShapeDtypeStruct((n, d), jnp.float32),
    )(alo, ahi, ylo, yhi, dinvb, b2)


def kernel(x, edge_index, W1, b1, W2, b2):
    n = x.shape[0]
    e = edge_index.shape[1]

    grp = NW * CHUNK * NB
    e_pad = ((e + grp - 1) // grp) * grp
    chw = e_pad // (NW * CHUNK)
    n_acc = ((n + 1 + NS * CHUNK - 1) // (NS * CHUNK)) * (NS * CHUNK)

    idt = edge_index.dtype
    src3 = jnp.concatenate(
        [edge_index[0], jnp.zeros((e_pad - e,), idt)]).reshape(NW, chw, CHUNK)
    dst3 = jnp.concatenate(
        [edge_index[1], jnp.full((e_pad - e,), n, idt)]).reshape(NW, chw, CHUNK)

    degp = _count_degrees(dst3, n_acc)
    y1lo, y1hi, dinvb = _first_layer(x, W1, degp[:, :n, :])
    a1lo, a1hi = _aggregate(y1lo, y1hi, src3, dst3, n_acc)
    y2lo, y2hi = _mid_layer(a1lo[:, :n, :], a1hi[:, :n, :], y1lo, y1hi,
                            dinvb, b1.reshape(1, -1), W2)
    a2lo, a2hi = _aggregate(y2lo, y2hi, src3, dst3, n_acc)
    return _final_layer(a2lo[:, :n, :], a2hi[:, :n, :], y2lo, y2hi,
                        dinvb, b2.reshape(1, -1))


# speedup vs baseline: 23.9946x; 2.8079x over previous
import jax
import jax.numpy as jnp
from jax import lax
from jax.experimental import pallas as pl
from jax.experimental.pallas import tpu as pltpu
from jax.experimental.pallas import tpu_sc as plsc

NC = 2
NS = 16
NW = NC * NS
LANE = 16
CHUNK = 128
DEGW = 16
NB = 4
KD = 8


def _mesh():
    return plsc.VectorSubcoreMesh(
        core_axis_name="c", subcore_axis_name="s", num_cores=NC, num_subcores=NS
    )


def _deg_body(dst3, out, idx_v, ones_v, zrow_v, acc, dsem):
    c = lax.axis_index("c")
    s = lax.axis_index("s")
    w = c * NS + s
    chw = idx_v.shape[0]
    rpt = zrow_v.shape[0]

    pltpu.sync_copy(dst3.at[w], idx_v)

    @pl.loop(0, CHUNK)
    def _(r):
        ones_v[r, :] = jnp.ones((DEGW,), jnp.float32)

    @pl.loop(0, rpt)
    def _(r):
        zrow_v[r, :] = jnp.zeros((DEGW,), jnp.float32)

    pltpu.sync_copy(zrow_v, acc.at[pl.ds(s * rpt, rpt)])
    plsc.subcore_barrier()

    for j in range(KD):
        pltpu.async_copy(ones_v, acc.at[idx_v.at[j]], dsem, add=True)

    @pl.loop(0, chw - KD)
    def _(j):
        pltpu.async_copy(ones_v, acc.at[idx_v.at[j + KD]], dsem, add=True)
        pltpu.make_async_copy(ones_v, acc.at[idx_v.at[0]], dsem).wait()

    for _j in range(KD):
        pltpu.make_async_copy(ones_v, acc.at[idx_v.at[0]], dsem).wait()

    plsc.subcore_barrier()
    pltpu.sync_copy(acc.at[pl.ds(s * rpt, rpt)], out.at[c, pl.ds(s * rpt, rpt)])


def _agg_body(y_lo, y_hi, src3, dst3, out_lo, out_hi, idxs_v, idxd_v, rows_v,
              acc, g0, g1, g2, g3, s0, s1, s2, s3):
    gsems = (g0, g1, g2, g3)
    ssems = (s0, s1, s2, s3)
    c = lax.axis_index("c")
    s = lax.axis_index("s")
    w = c * NS + s
    chw = idxs_v.shape[0]
    n_acc, dh = acc.shape
    rpt = n_acc // NS

    pltpu.sync_copy(src3.at[w], idxs_v)
    pltpu.sync_copy(dst3.at[w], idxd_v)

    def zero_acc_slice():
        for k in range(rpt // CHUNK):
            pltpu.sync_copy(rows_v.at[0], acc.at[pl.ds(s * rpt + k * CHUNK, CHUNK)])

    @pl.loop(0, CHUNK)
    def _(r):
        for q in range(dh // LANE):
            rows_v[0, r, pl.ds(q * LANE, LANE)] = jnp.zeros((LANE,), jnp.float32)

    def run_half(y, out):
        def start_gather(j, b):
            pltpu.async_copy(y.at[idxs_v.at[j]], rows_v.at[b], gsems[b])

        def wait_gather(b):
            pltpu.make_async_copy(y.at[idxs_v.at[0]], rows_v.at[b], gsems[b]).wait()

        def start_scatter(j, b):
            pltpu.async_copy(rows_v.at[b], acc.at[idxd_v.at[j]], ssems[b], add=True)

        def wait_scatter(b):
            pltpu.make_async_copy(
                rows_v.at[b], acc.at[idxd_v.at[0]], ssems[b]).wait()

        for b in range(NB):
            start_gather(b, b)

        @pl.loop(0, chw // NB - 1)
        def _(g):
            base = g * NB
            for b in range(NB):
                wait_gather(b)
                start_scatter(base + b, b)
            for b in range(NB):
                wait_scatter(b)
                start_gather(base + NB + b, b)

        last = chw - NB
        for b in range(NB):
            wait_gather(b)
            start_scatter(last + b, b)
        for b in range(NB):
            wait_scatter(b)

        plsc.subcore_barrier()
        for k in range(rpt // CHUNK):
            off = s * rpt + k * CHUNK
            pltpu.sync_copy(acc.at[pl.ds(off, CHUNK)], out.at[c, pl.ds(off, CHUNK)])

    zero_acc_slice()
    plsc.subcore_barrier()
    run_half(y_lo, out_lo)

    @pl.loop(0, CHUNK)
    def _(r):
        for q in range(dh // LANE):
            rows_v[0, r, pl.ds(q * LANE, LANE)] = jnp.zeros((LANE,), jnp.float32)

    zero_acc_slice()
    plsc.subcore_barrier()
    run_half(y_hi, out_hi)


_SC_PARAMS = pltpu.CompilerParams(use_tc_tiling_on_sc=False)


def _count_degrees(dst3, n_acc):
    chw = dst3.shape[1]
    rpt = n_acc // NS
    return pl.kernel(
        _deg_body,
        out_type=jax.ShapeDtypeStruct((NC, n_acc, DEGW), jnp.float32),
        mesh=_mesh(),
        compiler_params=_SC_PARAMS,
        scratch_types=[
            pltpu.VMEM((chw, CHUNK), jnp.int32),
            pltpu.VMEM((CHUNK, DEGW), jnp.float32),
            pltpu.VMEM((rpt, DEGW), jnp.float32),
            pltpu.VMEM_SHARED((n_acc, DEGW), jnp.float32),
            pltpu.SemaphoreType.DMA,
        ],
    )(dst3)


def _aggregate(y_lo, y_hi, src3, dst3, n_acc):
    chw = src3.shape[1]
    dh = y_lo.shape[1]
    out_t = jax.ShapeDtypeStruct((NC, n_acc, dh), jnp.float32)
    return pl.kernel(
        _agg_body,
        out_type=[out_t, out_t],
        mesh=_mesh(),
        compiler_params=_SC_PARAMS,
        scratch_types=[
            pltpu.VMEM((chw, CHUNK), jnp.int32),
            pltpu.VMEM((chw, CHUNK), jnp.int32),
            pltpu.VMEM((NB, CHUNK, dh), jnp.float32),
            pltpu.VMEM_SHARED((n_acc, dh), jnp.float32),
        ] + [pltpu.SemaphoreType.DMA] * (2 * NB),
    )(y_lo, y_hi, src3, dst3)


def _row_block(n):
    for bm in (1024, 1000, 800, 640, 512, 400, 256, 200, 128, 80, 40, 8):
        if n % bm == 0:
            return bm
    return n


def _first_layer(x, w1, degs):
    n, d_in = x.shape
    d_h = w1.shape[1]
    dh2 = d_h // 2
    bm = _row_block(n)

    def body(deg_ref, x_ref, w_ref, ylo_ref, yhi_ref, dinv_ref):
        deg = deg_ref[0, :, 0:1] + deg_ref[1, :, 0:1] + 1.0
        dinvb = jnp.broadcast_to(lax.rsqrt(deg), (bm, d_h))
        dinv_ref[...] = dinvb
        y = jnp.dot(
            x_ref[...], w_ref[...],
            preferred_element_type=jnp.float32,
            precision=lax.Precision.HIGHEST,
        ) * dinvb
        ylo_ref[...] = y[:, :dh2]
        yhi_ref[...] = y[:, dh2:]

    return pl.pallas_call(
        body,
        grid=(n // bm,),
        in_specs=[
            pl.BlockSpec((NC, bm, DEGW), lambda i: (0, i, 0)),
            pl.BlockSpec((bm, d_in), lambda i: (i, 0)),
            pl.BlockSpec((d_in, d_h), lambda i: (0, 0)),
        ],
        out_specs=[
            pl.BlockSpec((bm, dh2), lambda i: (i, 0)),
            pl.BlockSpec((bm, dh2), lambda i: (i, 0)),
            pl.BlockSpec((bm, d_h), lambda i: (i, 0)),
        ],
        out_shape=[
            jax.ShapeDtypeStruct((n, dh2), jnp.float32),
            jax.ShapeDtypeStruct((n, dh2), jnp.float32),
            jax.ShapeDtypeStruct((n, d_h), jnp.float32),
        ],
    )(degs, x, w1)


def _mid_layer(alo, ahi, ylo, yhi, dinvb, b1, w2):
    n, d = dinvb.shape
    dh2 = d // 2
    bm = _row_block(n)

    def body(alo_ref, ahi_ref, ylo_ref, yhi_ref, dinv_ref, b_ref, w_ref,
             y2lo_ref, y2hi_ref):
        dinv = dinv_ref[...]
        h_lo = dinv[:, :dh2] * (alo_ref[0] + alo_ref[1] + ylo_ref[...]) \
            + b_ref[:, :dh2]
        h_hi = dinv[:, dh2:] * (ahi_ref[0] + ahi_ref[1] + yhi_ref[...]) \
            + b_ref[:, dh2:]
        h = jnp.maximum(jnp.concatenate([h_lo, h_hi], axis=-1), 0.0)
        y2 = jnp.dot(
            h, w_ref[...],
            preferred_element_type=jnp.float32,
            precision=lax.Precision.HIGHEST,
        ) * dinv
        y2lo_ref[...] = y2[:, :dh2]
        y2hi_ref[...] = y2[:, dh2:]

    half = pl.BlockSpec((bm, dh2), lambda i: (i, 0))
    part = pl.BlockSpec((NC, bm, dh2), lambda i: (0, i, 0))
    return pl.pallas_call(
        body,
        grid=(n // bm,),
        in_specs=[
            part, part, half, half,
            pl.BlockSpec((bm, d), lambda i: (i, 0)),
            pl.BlockSpec((1, d), lambda i: (0, 0)),
            pl.BlockSpec((d, d), lambda i: (0, 0)),
        ],
        out_specs=[half, half],
        out_shape=[
            jax.ShapeDtypeStruct((n, dh2), jnp.float32),
            jax.ShapeDtypeStruct((n, dh2), jnp.float32),
        ],
    )(alo, ahi, ylo, yhi, dinvb, b1, w2)


def _final_layer(alo, ahi, ylo, yhi, dinvb, b2):
    n, d = dinvb.shape
    dh2 = d // 2
    bm = _row_block(n)

    def body(alo_ref, ahi_ref, ylo_ref, yhi_ref, dinv_ref, b_ref, h_ref):
        dinv = dinv_ref[...]
        h_lo = dinv[:, :dh2] * (alo_ref[0] + alo_ref[1] + ylo_ref[...]) \
            + b_ref[:, :dh2]
        h_hi = dinv[:, dh2:] * (ahi_ref[0] + ahi_ref[1] + yhi_ref[...]) \
            + b_ref[:, dh2:]
        h_ref[...] = jnp.maximum(jnp.concatenate([h_lo, h_hi], axis=-1), 0.0)

    half = pl.BlockSpec((bm, dh2), lambda i: (i, 0))
    part = pl.BlockSpec((NC, bm, dh2), lambda i: (0, i, 0))
    return pl.pallas_call(
        body,
        grid=(n // bm,),
        in_specs=[
            part, part, half, half,
            pl.BlockSpec((bm, d), lambda i: (i, 0)),
            pl.BlockSpec((1, d), lambda i: (0, 0)),
        ],
        out_specs=pl.BlockSpec((bm, d), lambda i: (i, 0)),
        out_shape=jax.ShapeDtypeStruct((n, d), jnp.float32),
    )(alo, ahi, ylo, yhi, dinvb, b2)


def kernel(x, edge_index, W1, b1, W2, b2):
    n = x.shape[0]
    e = edge_index.shape[1]

    grp = NW * CHUNK * NB
    e_pad = ((e + grp - 1) // grp) * grp
    chw = e_pad // (NW * CHUNK)
    n_acc = ((n + 1 + NS * CHUNK - 1) // (NS * CHUNK)) * (NS * CHUNK)

    idt = edge_index.dtype
    pad_i = jnp.arange(e_pad - e, dtype=idt)
    src3 = jnp.concatenate(
        [edge_index[0], pad_i % n]).reshape(NW, chw, CHUNK)
    dst3 = jnp.concatenate(
        [edge_index[1], n + pad_i % (n_acc - n)]).reshape(NW, chw, CHUNK)

    degp = _count_degrees(dst3, n_acc)
    y1lo, y1hi, dinvb = _first_layer(x, W1, degp[:, :n, :])
    a1lo, a1hi = _aggregate(y1lo, y1hi, src3, dst3, n_acc)
    y2lo, y2hi = _mid_layer(a1lo[:, :n, :], a1hi[:, :n, :], y1lo, y1hi,
                            dinvb, b1.reshape(1, -1), W2)
    a2lo, a2hi = _aggregate(y2lo, y2hi, src3, dst3, n_acc)
    return _final_layer(a2lo[:, :n, :], a2hi[:, :n, :], y2lo, y2hi,
                        dinvb, b2.reshape(1, -1))


# speedup vs baseline: 26.1524x; 1.0899x over previous
import jax
import jax.numpy as jnp
from jax import lax
from jax.experimental import pallas as pl
from jax.experimental.pallas import tpu as pltpu
from jax.experimental.pallas import tpu_sc as plsc

NC = 2
NS = 16
NW = NC * NS
LANE = 16
CHUNK = 128
DEGW = 16
NB = 4
KD = 8


def _mesh():
    return plsc.VectorSubcoreMesh(
        core_axis_name="c", subcore_axis_name="s", num_cores=NC, num_subcores=NS
    )


def _deg_body(dst3, out, idx_v, ones_v, zrow_v, acc, dsem):
    c = lax.axis_index("c")
    s = lax.axis_index("s")
    w = c * NS + s
    chw = idx_v.shape[0]
    rpt = zrow_v.shape[0]

    pltpu.sync_copy(dst3.at[w], idx_v)

    @pl.loop(0, CHUNK)
    def _(r):
        ones_v[r, :] = jnp.ones((DEGW,), jnp.float32)

    @pl.loop(0, rpt)
    def _(r):
        zrow_v[r, :] = jnp.zeros((DEGW,), jnp.float32)

    pltpu.sync_copy(zrow_v, acc.at[pl.ds(s * rpt, rpt)])
    plsc.subcore_barrier()

    for j in range(KD):
        pltpu.async_copy(ones_v, acc.at[idx_v.at[j]], dsem, add=True)

    @pl.loop(0, chw - KD)
    def _(j):
        pltpu.async_copy(ones_v, acc.at[idx_v.at[j + KD]], dsem, add=True)
        pltpu.make_async_copy(ones_v, acc.at[idx_v.at[0]], dsem).wait()

    for _j in range(KD):
        pltpu.make_async_copy(ones_v, acc.at[idx_v.at[0]], dsem).wait()

    plsc.subcore_barrier()
    pltpu.sync_copy(acc.at[pl.ds(s * rpt, rpt)], out.at[c, pl.ds(s * rpt, rpt)])


def _agg_body(y_lo, y_hi, src3, dst3, out_lo, out_hi, idxs_v, idxd_v, rows_v,
              acc, g0, g1, g2, g3, s0, s1, s2, s3):
    gsems = (g0, g1, g2, g3)
    ssems = (s0, s1, s2, s3)
    c = lax.axis_index("c")
    s = lax.axis_index("s")
    w = c * NS + s
    chw = idxs_v.shape[0]
    n_acc, dh = acc.shape
    rpt = n_acc // NS

    pltpu.sync_copy(src3.at[w], idxs_v)
    pltpu.sync_copy(dst3.at[w], idxd_v)

    def zero_acc_slice():
        for k in range(rpt // CHUNK):
            pltpu.sync_copy(rows_v.at[0], acc.at[pl.ds(s * rpt + k * CHUNK, CHUNK)])

    @pl.loop(0, CHUNK)
    def _(r):
        for q in range(dh // LANE):
            rows_v[0, r, pl.ds(q * LANE, LANE)] = jnp.zeros((LANE,), jnp.float32)

    def run_half(y, out):
        def start_gather(j, b):
            pltpu.async_copy(y.at[idxs_v.at[j]], rows_v.at[b], gsems[b])

        def wait_gather(b):
            pltpu.make_async_copy(y.at[idxs_v.at[0]], rows_v.at[b], gsems[b]).wait()

        def start_scatter(j, b):
            pltpu.async_copy(rows_v.at[b], acc.at[idxd_v.at[j]], ssems[b], add=True)

        def wait_scatter(b):
            pltpu.make_async_copy(
                rows_v.at[b], acc.at[idxd_v.at[0]], ssems[b]).wait()

        for b in range(NB):
            start_gather(b, b)

        @pl.loop(0, chw // NB - 1)
        def _(g):
            base = g * NB
            for b in range(NB):
                wait_gather(b)
                start_scatter(base + b, b)
            for b in range(NB):
                wait_scatter(b)
                start_gather(base + NB + b, b)

        last = chw - NB
        for b in range(NB):
            wait_gather(b)
            start_scatter(last + b, b)
        for b in range(NB):
            wait_scatter(b)

        plsc.subcore_barrier()
        for k in range(rpt // CHUNK):
            off = s * rpt + k * CHUNK
            pltpu.sync_copy(acc.at[pl.ds(off, CHUNK)], out.at[c, pl.ds(off, CHUNK)])

    zero_acc_slice()
    plsc.subcore_barrier()
    run_half(y_lo, out_lo)

    @pl.loop(0, CHUNK)
    def _(r):
        for q in range(dh // LANE):
            rows_v[0, r, pl.ds(q * LANE, LANE)] = jnp.zeros((LANE,), jnp.float32)

    zero_acc_slice()
    plsc.subcore_barrier()
    run_half(y_hi, out_hi)


_SC_PARAMS = pltpu.CompilerParams(use_tc_tiling_on_sc=False)


def _count_degrees(dst3, n_acc):
    chw = dst3.shape[1]
    rpt = n_acc // NS
    return pl.kernel(
        _deg_body,
        out_type=jax.ShapeDtypeStruct((NC, n_acc, DEGW), jnp.float32),
        mesh=_mesh(),
        compiler_params=_SC_PARAMS,
        scratch_types=[
            pltpu.VMEM((chw, CHUNK), jnp.int32),
            pltpu.VMEM((CHUNK, DEGW), jnp.float32),
            pltpu.VMEM((rpt, DEGW), jnp.float32),
            pltpu.VMEM_SHARED((n_acc, DEGW), jnp.float32),
            pltpu.SemaphoreType.DMA,
        ],
    )(dst3)


def _aggregate(y_lo, y_hi, src3, dst3, n_acc):
    chw = src3.shape[1]
    dh = y_lo.shape[1]
    out_t = jax.ShapeDtypeStruct((NC, n_acc, dh), jnp.float32)
    return pl.kernel(
        _agg_body,
        out_type=[out_t, out_t],
        mesh=_mesh(),
        compiler_params=_SC_PARAMS,
        scratch_types=[
            pltpu.VMEM((chw, CHUNK), jnp.int32),
            pltpu.VMEM((chw, CHUNK), jnp.int32),
            pltpu.VMEM((NB, CHUNK, dh), jnp.float32),
            pltpu.VMEM_SHARED((n_acc, dh), jnp.float32),
        ] + [pltpu.SemaphoreType.DMA] * (2 * NB),
    )(y_lo, y_hi, src3, dst3)


def _row_block(n):
    for bm in (1024, 1000, 800, 640, 512, 400, 256, 200, 128, 80, 40, 8):
        if n % bm == 0:
            return bm
    return n


def _first_layer(x, w1, degs):
    n, d_in = x.shape
    d_h = w1.shape[1]
    dh2 = d_h // 2
    bm = _row_block(n)

    def body(deg_ref, x_ref, w_ref, ylo_ref, yhi_ref, dinv_ref):
        deg = deg_ref[0, :, 0:1] + deg_ref[1, :, 0:1] + 1.0
        dinvb = jnp.broadcast_to(lax.rsqrt(deg), (bm, d_h))
        dinv_ref[...] = dinvb
        y = jnp.dot(
            x_ref[...], w_ref[...],
            preferred_element_type=jnp.float32,
            precision=lax.Precision.HIGHEST,
        ) * dinvb
        ylo_ref[...] = y[:, :dh2]
        yhi_ref[...] = y[:, dh2:]

    return pl.pallas_call(
        body,
        grid=(n // bm,),
        in_specs=[
            pl.BlockSpec((NC, bm, DEGW), lambda i: (0, i, 0)),
            pl.BlockSpec((bm, d_in), lambda i: (i, 0)),
            pl.BlockSpec((d_in, d_h), lambda i: (0, 0)),
        ],
        out_specs=[
            pl.BlockSpec((bm, dh2), lambda i: (i, 0)),
            pl.BlockSpec((bm, dh2), lambda i: (i, 0)),
            pl.BlockSpec((bm, d_h), lambda i: (i, 0)),
        ],
        out_shape=[
            jax.ShapeDtypeStruct((n, dh2), jnp.float32),
            jax.ShapeDtypeStruct((n, dh2), jnp.float32),
            jax.ShapeDtypeStruct((n, d_h), jnp.float32),
        ],
    )(degs, x, w1)


def _mid_layer(alo, ahi, ylo, yhi, dinvb, b1, w2):
    n, d = dinvb.shape
    dh2 = d // 2
    bm = _row_block(n)

    def body(alo_ref, ahi_ref, ylo_ref, yhi_ref, dinv_ref, b_ref, w_ref,
             y2lo_ref, y2hi_ref):
        dinv = dinv_ref[...]
        h_lo = dinv[:, :dh2] * (alo_ref[0] + alo_ref[1] + ylo_ref[...]) \
            + b_ref[:, :dh2]
        h_hi = dinv[:, dh2:] * (ahi_ref[0] + ahi_ref[1] + yhi_ref[...]) \
            + b_ref[:, dh2:]
        h = jnp.maximum(jnp.concatenate([h_lo, h_hi], axis=-1), 0.0)
        y2 = jnp.dot(
            h, w_ref[...],
            preferred_element_type=jnp.float32,
            precision=lax.Precision.HIGHEST,
        ) * dinv
        y2lo_ref[...] = y2[:, :dh2]
        y2hi_ref[...] = y2[:, dh2:]

    half = pl.BlockSpec((bm, dh2), lambda i: (i, 0))
    part = pl.BlockSpec((NC, bm, dh2), lambda i: (0, i, 0))
    return pl.pallas_call(
        body,
        grid=(n // bm,),
        in_specs=[
            part, part, half, half,
            pl.BlockSpec((bm, d), lambda i: (i, 0)),
            pl.BlockSpec((1, d), lambda i: (0, 0)),
            pl.BlockSpec((d, d), lambda i: (0, 0)),
        ],
        out_specs=[half, half],
        out_shape=[
            jax.ShapeDtypeStruct((n, dh2), jnp.float32),
            jax.ShapeDtypeStruct((n, dh2), jnp.float32),
        ],
    )(alo, ahi, ylo, yhi, dinvb, b1, w2)


def _final_layer(alo, ahi, ylo, yhi, dinvb, b2):
    n, d = dinvb.shape
    dh2 = d // 2
    bm = _row_block(n)

    def body(alo_ref, ahi_ref, ylo_ref, yhi_ref, dinv_ref, b_ref, h_ref):
        dinv = dinv_ref[...]
        h_lo = dinv[:, :dh2] * (alo_ref[0] + alo_ref[1] + ylo_ref[...]) \
            + b_ref[:, :dh2]
        h_hi = dinv[:, dh2:] * (ahi_ref[0] + ahi_ref[1] + yhi_ref[...]) \
            + b_ref[:, dh2:]
        h_ref[...] = jnp.maximum(jnp.concatenate([h_lo, h_hi], axis=-1), 0.0)

    half = pl.BlockSpec((bm, dh2), lambda i: (i, 0))
    part = pl.BlockSpec((NC, bm, dh2), lambda i: (0, i, 0))
    return pl.pallas_call(
        body,
        grid=(n // bm,),
        in_specs=[
            part, part, half, half,
            pl.BlockSpec((bm, d), lambda i: (i, 0)),
            pl.BlockSpec((1, d), lambda i: (0, 0)),
        ],
        out_specs=pl.BlockSpec((bm, d), lambda i: (i, 0)),
        out_shape=jax.ShapeDtypeStruct((n, d), jnp.float32),
    )(alo, ahi, ylo, yhi, dinvb, b2)


def kernel(x, edge_index, W1, b1, W2, b2):
    n = x.shape[0]
    e = edge_index.shape[1]

    grp = NW * CHUNK * NB
    e_pad = ((e + grp - 1) // grp) * grp
    chw = e_pad // (NW * CHUNK)
    n_acc = ((n + 1 + NS * CHUNK - 1) // (NS * CHUNK)) * (NS * CHUNK)

    idt = edge_index.dtype
    pad_i = jnp.arange(e_pad - e, dtype=idt)
    src3 = jnp.concatenate(
        [edge_index[0], pad_i % n]).reshape(NW, chw, CHUNK)
    dst3 = jnp.concatenate(
        [edge_index[1], n + pad_i % (n_acc - n)]).reshape(NW, chw, CHUNK)

    degp = _count_degrees(dst3, n_acc)
    y1lo, y1hi, dinvb = _first_layer(x, W1, degp)
    a1lo, a1hi = _aggregate(y1lo, y1hi, src3, dst3, n_acc)
    y2lo, y2hi = _mid_layer(a1lo, a1hi, y1lo, y1hi,
                            dinvb, b1.reshape(1, -1), W2)
    a2lo, a2hi = _aggregate(y2lo, y2hi, src3, dst3, n_acc)
    return _final_layer(a2lo, a2hi, y2lo, y2hi,
                        dinvb, b2.reshape(1, -1))
